# baseline (speedup 1.0000x reference)
import functools

import jax
import jax.numpy as jnp
from jax import lax
from jax.experimental import pallas as pl
from jax.experimental.pallas import tpu as pltpu
from jax.experimental.pallas import tpu_sc as plsc

NV = 4096
D = 256
DEG = 16

NC = 2
NS = 16
L = 16
NW = NC * NS
NPW = NV // NW
CH = 8
RPC = CH * DEG
NCHUNK = NPW // CH


def _sc_bf16(x):
  def rne(v):
    u = lax.bitcast_convert_type(v, jnp.int32)
    rb = lax.bitwise_and(lax.shift_right_logical(u, 16), 1)
    return lax.shift_right_logical(u + 32767 + rb, 16)

  return lax.bitwise_or(rne(x[:, :D // 2]),
                        lax.shift_left(rne(x[:, D // 2:]), 16))


@functools.lru_cache(maxsize=None)
def _make_sc_agg(with_addend: bool, scale: float):
  mesh = plsc.VectorSubcoreMesh(
      core_axis_name="c", subcore_axis_name="s", num_cores=NC, num_subcores=NS)
  out_type = (jax.ShapeDtypeStruct((NV, D), jnp.float32),
              jax.ShapeDtypeStruct((NV, D), jnp.float32))
  scratch = [
      pltpu.VMEM((NPW * DEG,), jnp.int32),
      pltpu.VMEM((2, RPC, D // 2), jnp.int32),
      pltpu.VMEM((2, CH, D), jnp.float32),
      pltpu.SemaphoreType.DMA,
      pltpu.SemaphoreType.DMA,
      pltpu.SemaphoreType.DMA,
      pltpu.SemaphoreType.DMA,
  ]
  if with_addend:
    scratch.append(pltpu.VMEM((NPW, D), jnp.float32))

  def body(*refs):
    if with_addend:
      (ta, ia, tb, ib, ada, adb, outa, outb,
       idx_v, rows_v, out_v, gsem0, gsem1, osem0, osem1, add_v) = refs
      sides = ((ta, ia, ada, outa), (tb, ib, adb, outb))
    else:
      (ta, ia, tb, ib, outa, outb,
       idx_v, rows_v, out_v, gsem0, gsem1, osem0, osem1) = refs
      add_v = None
      sides = ((ta, ia, None, outa), (tb, ib, None, outb))
    gsems = (gsem0, gsem1)
    osems = (osem0, osem1)
    wid = lax.axis_index("s") * NC + lax.axis_index("c")
    node0 = wid * NPW

    for table, idx, addend, out in sides:
      pltpu.sync_copy(idx.at[pl.ds(node0 * DEG, NPW * DEG)], idx_v)
      if addend is not None:
        pltpu.sync_copy(addend.at[pl.ds(node0, NPW)], add_v)

      def start_gather(c, b, table=table):
        pltpu.async_copy(
            table.at[idx_v.at[pl.ds(c * RPC, RPC)]], rows_v.at[b], gsems[b])

      def wait_gather(b, table=table):
        pltpu.make_async_copy(
            table.at[pl.ds(0, RPC)], rows_v.at[b], gsems[b]).wait()

      def wait_out(b, out=out):
        pltpu.make_async_copy(
            out_v.at[b], out.at[pl.ds(node0, CH)], osems[b]).wait()

      def reduce_chunk(c, b, addend=addend):
        mask = jnp.int32(-65536)

        def split(w):
          e = lax.bitcast_convert_type(lax.shift_left(w, 16), jnp.float32)
          o = lax.bitcast_convert_type(lax.bitwise_and(w, mask), jnp.float32)
          return e, o

        def lane_body(g, _):
          sl = pl.ds(g * 16, 16)
          slo = pl.ds(g * 16, 16)
          shi = pl.ds(D // 2 + g * 16, 16)
          for cc in range(CH):
            e, o = split(rows_v[b, cc * DEG, sl])
            for r in range(1, DEG):
              e2, o2 = split(rows_v[b, cc * DEG + r, sl])
              e = e + e2
              o = o + o2
            if addend is not None:
              e = e + add_v[c * CH + cc, slo]
              o = o + add_v[c * CH + cc, shi]
            out_v[b, cc, slo] = e * scale
            out_v[b, cc, shi] = o * scale
          return 0
        lax.fori_loop(0, D // 32, lane_body, 0)

      start_gather(0, 0)

      def pair_body(kk, _, out=out):
        for b in range(2):
          c = kk * 2 + b
          wait_gather(b)

          @pl.when(c + 1 < NCHUNK)
          def _():
            start_gather(c + 1, 1 - b)

          @pl.when(c >= 2)
          def _():
            wait_out(b)

          reduce_chunk(c, b)
          pltpu.async_copy(out_v.at[b],
                           out.at[pl.ds(node0 + c * CH, CH)], osems[b])
        return 0

      lax.fori_loop(0, NCHUNK // 2, pair_body, 0)
      for b in range(2):
        wait_out(b)

  return pl.kernel(body, out_type=out_type, mesh=mesh, scratch_types=scratch)


def _sc_agg_mean(*args):
  return _make_sc_agg(False, 1.0 / DEG)(*args)


def _sc_agg_sum(*args):
  return _make_sc_agg(False, 1.0)(*args)


BM = 512
NBLK = NV // BM


def _prelu(y, a):
  return jnp.where(y >= 0, y, a * y)


def _cast_body(xv, xu, ov, ou):
  ov[...] = _sc_bf16(xv[...])
  ou[...] = _sc_bf16(xu[...])


def _stage1_body(xv, xu, wv, wu, bv, bu, a_ref, ov, ou):
  a = a_ref[0]
  ev = _prelu(jnp.dot(xv[...], wv[...], preferred_element_type=jnp.float32)
              + bv[...], a)
  eu = _prelu(jnp.dot(xu[...], wu[...], preferred_element_type=jnp.float32)
              + bu[...], a)
  ov[...] = _sc_bf16(ev)
  ou[...] = _sc_bf16(eu)


def _stage2_body(xv, xu, wv, wu, bv, bu, a_ref, wd, ov, ou, oa, oub, ovp, oup):
  a = a_ref[0]
  ev = _prelu(jnp.dot(xv[...], wv[...], preferred_element_type=jnp.float32)
              + bv[...], a)
  eu = _prelu(jnp.dot(xu[...], wu[...], preferred_element_type=jnp.float32)
              + bu[...], a)
  oa[...] = jnp.dot(ev, wd[...],
                    preferred_element_type=jnp.float32).astype(jnp.bfloat16)
  ov[...] = ev
  ou[...] = eu
  oub[...] = eu.astype(jnp.bfloat16)
  ovp[...] = _sc_bf16(ev)
  oup[...] = _sc_bf16(eu)


def _stage3_body(ev2, eu2, fv, fu, wv3, wu3, bv3, bu3, ov, ou):
  ov[...] = (jnp.dot(ev2[...], wv3[:D, :], preferred_element_type=jnp.float32)
             + jnp.dot(fv[...], wv3[D:, :], preferred_element_type=jnp.float32)
             + bv3[...])
  ou[...] = (jnp.dot(eu2[...], wu3[:D, :], preferred_element_type=jnp.float32)
             + jnp.dot(fu[...], wu3[D:, :], preferred_element_type=jnp.float32)
             + bu3[...])


def _final_body(gv, gu, av1, au1, v3, u3, wv3, wu3, bv3, bu3, osv, osu):
  c = jnp.float32(1.0 / (DEG + 1.0))
  d = jnp.float32(float(DEG))
  osv[...] = (jnp.dot(gv[...], wu3[:D, :], preferred_element_type=jnp.float32)
              + d * jnp.dot(av1[...], wu3[D:, :],
                            preferred_element_type=jnp.float32)
              + d * bu3[...] + v3[...]) * c
  osu[...] = (jnp.dot(gu[...], wv3[:D, :], preferred_element_type=jnp.float32)
              + d * jnp.dot(au1[...], wv3[D:, :],
                            preferred_element_type=jnp.float32)
              + d * bv3[...] + u3[...]) * c


def _row_spec():
  return pl.BlockSpec((BM, D), lambda i: (i, 0))


def _packed_spec():
  return pl.BlockSpec((BM, D // 2), lambda i: (i, 0))


def _full_spec():
  return pl.BlockSpec((D, D), lambda i: (0, 0))


def _bias_spec():
  return pl.BlockSpec((1, D), lambda i: (0, 0))


_f32_row = jax.ShapeDtypeStruct((NV, D), jnp.float32)
_bf16_row = jax.ShapeDtypeStruct((NV, D // 2), jnp.int32)

_cast_call = pl.pallas_call(
    _cast_body,
    grid=(NBLK,),
    in_specs=[_row_spec(), _row_spec()],
    out_specs=[_packed_spec(), _packed_spec()],
    out_shape=[_bf16_row, _bf16_row],
)

_stage1_call = pl.pallas_call(
    _stage1_body,
    grid=(NBLK,),
    in_specs=[_row_spec(), _row_spec(), _full_spec(), _full_spec(),
              _bias_spec(), _bias_spec(),
              pl.BlockSpec(memory_space=pltpu.SMEM)],
    out_specs=[_packed_spec(), _packed_spec()],
    out_shape=[_bf16_row, _bf16_row],
)

_stage2_call = pl.pallas_call(
    _stage2_body,
    grid=(NBLK,),
    in_specs=[_row_spec(), _row_spec(), _full_spec(), _full_spec(),
              _bias_spec(), _bias_spec(),
              pl.BlockSpec(memory_space=pltpu.SMEM), _full_spec()],
    out_specs=[_row_spec(), _row_spec(), _row_spec(), _row_spec(),
               _packed_spec(), _packed_spec()],
    out_shape=[_f32_row, _f32_row,
               jax.ShapeDtypeStruct((NV, D), jnp.bfloat16),
               jax.ShapeDtypeStruct((NV, D), jnp.bfloat16),
               _bf16_row, _bf16_row],
)

_stage3_call = pl.pallas_call(
    _stage3_body,
    grid=(NBLK,),
    in_specs=[_row_spec(), _row_spec(), _row_spec(), _row_spec(),
              pl.BlockSpec((2 * D, D), lambda i: (0, 0)),
              pl.BlockSpec((2 * D, D), lambda i: (0, 0)),
              _bias_spec(), _bias_spec()],
    out_specs=[_row_spec(), _row_spec()],
    out_shape=[_f32_row, _f32_row],
)

_final_call = pl.pallas_call(
    _final_body,
    grid=(NBLK,),
    in_specs=[_row_spec(), _row_spec(), _row_spec(), _row_spec(),
              _row_spec(), _row_spec(),
              pl.BlockSpec((2 * D, D), lambda i: (0, 0)),
              pl.BlockSpec((2 * D, D), lambda i: (0, 0)),
              _bias_spec(), _bias_spec()],
    out_specs=[_row_spec(), _row_spec()],
    out_shape=[_f32_row, _f32_row],
)


LBM = 512
LBN = 1024
LNI = NV // LBM
LNJ = NV // LBN


def _loss_body(a_ref, u_ref, t_ref, o_ref, acc_ref):
  i = pl.program_id(0)
  j = pl.program_id(1)

  @pl.when((i == 0) & (j == 0))
  def _():
    acc_ref[0] = 0.0
    acc_ref[1] = 0.0
    acc_ref[2] = 0.0
    acc_ref[3] = 0.0

  logit = lax.dot_general(a_ref[...], u_ref[...], (((1,), (1,)), ((), ())),
                          preferred_element_type=jnp.float32)
  t = t_ref[...].astype(jnp.float32)
  sp = jnp.maximum(logit, 0.0) + jnp.log1p(jnp.exp(-jnp.abs(logit)))
  acc_ref[0] += jnp.sum(sp)
  acc_ref[1] += jnp.sum(t * (sp - logit))
  acc_ref[2] += jnp.sum(t * sp)
  acc_ref[3] += jnp.sum(t)

  @pl.when((i == LNI - 1) & (j == LNJ - 1))
  def _():
    n = float(NV) * float(NV)
    s = acc_ref[3]
    norm = n / (n - s)
    pw = (n - s) / s
    val = (norm / n) * (acc_ref[0] + pw * acc_ref[1] - acc_ref[2])
    o_ref[...] = jnp.reshape(val, (1, 1))


_loss_call = pl.pallas_call(
    _loss_body,
    grid=(LNI, LNJ),
    in_specs=[pl.BlockSpec((LBM, D), lambda i, j: (i, 0)),
              pl.BlockSpec((LBN, D), lambda i, j: (j, 0)),
              pl.BlockSpec((LBM, LBN), lambda i, j: (i, j))],
    out_specs=pl.BlockSpec((1, 1), lambda i, j: (0, 0)),
    out_shape=jax.ShapeDtypeStruct((1, 1), jnp.float32),
    scratch_shapes=[pltpu.SMEM((4,), jnp.float32)],
    compiler_params=pltpu.CompilerParams(
        dimension_semantics=("arbitrary", "arbitrary")),
)


def kernel(feat_v, feat_u, neigh_v, neigh_u, target,
           Wv1, bv1, Wu1, bu1, a1, Wv2, bv2, Wu2, bu2, a2,
           Wv3, bv3, Wu3, bu3, Wd):
  iv = neigh_v.reshape(-1)
  iu = neigh_u.reshape(-1)

  fvb, fub = _cast_call(feat_v, feat_u)
  aggv1, aggu1 = _sc_agg_mean(fub, iv, fvb, iu)
  ve1b, ue1b = _stage1_call(aggv1, aggu1, Wv1, Wu1,
                            bv1.reshape(1, D), bu1.reshape(1, D),
                            a1.reshape(1))
  aggv2, aggu2 = _sc_agg_mean(ue1b, iv, ve1b, iu)
  ve2, ue2, Ab, ue2b, ve2p, ue2p = _stage2_call(
      aggv2, aggu2, Wv2, Wu2, bv2.reshape(1, D), bu2.reshape(1, D),
      a2.reshape(1), Wd)
  gv, gu = _sc_agg_sum(ue2p, iv, ve2p, iu)
  ve3, ue3 = _stage3_call(ve2, ue2, feat_v, feat_u, Wv3, Wu3,
                          bv3.reshape(1, D), bu3.reshape(1, D))
  loss = _loss_call(Ab, ue2b, target)[0, 0]
  sv, su = _final_call(gv, gu, aggv1, aggu1, ve3, ue3, Wv3, Wu3,
                       bv3.reshape(1, D), bu3.reshape(1, D))
  return ve2, ue2, sv, su, loss


# speedup vs baseline: 1.0556x; 1.0556x over previous
import functools

import jax
import jax.numpy as jnp
from jax import lax
from jax.experimental import pallas as pl
from jax.experimental.pallas import tpu as pltpu
from jax.experimental.pallas import tpu_sc as plsc

NV = 4096
D = 256
DEG = 16

NC = 2
NS = 16
L = 16
NW = NC * NS
NPW = NV // NW
CH = 8
RPC = CH * DEG
NCHUNK = NPW // CH


def _sc_bf16(x):
  def rne(v):
    u = lax.bitcast_convert_type(v, jnp.int32)
    rb = lax.bitwise_and(lax.shift_right_logical(u, 16), 1)
    return lax.shift_right_logical(u + 32767 + rb, 16)

  return lax.bitwise_or(rne(x[:, :D // 2]),
                        lax.shift_left(rne(x[:, D // 2:]), 16))


@functools.lru_cache(maxsize=None)
def _make_sc_agg(scale: float):
  mesh = plsc.VectorSubcoreMesh(
      core_axis_name="c", subcore_axis_name="s", num_cores=NC, num_subcores=NS)
  out_type = (jax.ShapeDtypeStruct((NV, D), jnp.float32),
              jax.ShapeDtypeStruct((NV, D), jnp.float32))
  scratch = [
      pltpu.VMEM((NPW * DEG,), jnp.int32),
      pltpu.VMEM((2, RPC, D // 2), jnp.int32),
      pltpu.VMEM((2, CH, D), jnp.float32),
      pltpu.SemaphoreType.DMA,
      pltpu.SemaphoreType.DMA,
      pltpu.SemaphoreType.DMA,
      pltpu.SemaphoreType.DMA,
  ]
  def body(*refs):
    (ta, ia, tb, ib, outa, outb,
     idx_v, rows_v, out_v, gsem0, gsem1, osem0, osem1) = refs
    sides = ((ta, ia, outa), (tb, ib, outb))
    gsems = (gsem0, gsem1)
    osems = (osem0, osem1)
    wid = lax.axis_index("s") * NC + lax.axis_index("c")
    node0 = wid * NPW

    for table, idx, out in sides:
      pltpu.sync_copy(idx.at[pl.ds(node0 * DEG, NPW * DEG)], idx_v)

      def start_gather(c, b, table=table):
        pltpu.async_copy(
            table.at[idx_v.at[pl.ds(c * RPC, RPC)]], rows_v.at[b], gsems[b])

      def wait_gather(b, table=table):
        pltpu.make_async_copy(
            table.at[pl.ds(0, RPC)], rows_v.at[b], gsems[b]).wait()

      def wait_out(b, out=out):
        pltpu.make_async_copy(
            out_v.at[b], out.at[pl.ds(node0, CH)], osems[b]).wait()

      def reduce_chunk(c, b):
        mask = jnp.int32(-65536)

        def split(w):
          e = lax.bitcast_convert_type(lax.shift_left(w, 16), jnp.float32)
          o = lax.bitcast_convert_type(lax.bitwise_and(w, mask), jnp.float32)
          return e, o

        def lane_body(g, _):
          sl = pl.ds(g * 16, 16)
          slo = pl.ds(g * 16, 16)
          shi = pl.ds(D // 2 + g * 16, 16)
          for cc in range(CH):
            e, o = split(rows_v[b, cc * DEG, sl])
            for r in range(1, DEG):
              e2, o2 = split(rows_v[b, cc * DEG + r, sl])
              e = e + e2
              o = o + o2
            out_v[b, cc, slo] = e * scale
            out_v[b, cc, shi] = o * scale
          return 0
        lax.fori_loop(0, D // 32, lane_body, 0)

      start_gather(0, 0)

      def pair_body(kk, _, out=out):
        for b in range(2):
          c = kk * 2 + b
          wait_gather(b)

          @pl.when(c + 1 < NCHUNK)
          def _():
            start_gather(c + 1, 1 - b)

          @pl.when(c >= 2)
          def _():
            wait_out(b)

          reduce_chunk(c, b)
          pltpu.async_copy(out_v.at[b],
                           out.at[pl.ds(node0 + c * CH, CH)], osems[b])
        return 0

      lax.fori_loop(0, NCHUNK // 2, pair_body, 0)
      for b in range(2):
        wait_out(b)

  return pl.kernel(body, out_type=out_type, mesh=mesh, scratch_types=scratch)


def _sc_agg_mean(*args):
  return _make_sc_agg(1.0 / DEG)(*args)


def _sc_agg_sum(*args):
  return _make_sc_agg(1.0)(*args)


BM = 512
NBLK = NV // BM


def _prelu(y, a):
  return jnp.where(y >= 0, y, a * y)


def _cast_body(xv, xu, ov, ou):
  ov[...] = _sc_bf16(xv[...])
  ou[...] = _sc_bf16(xu[...])


def _stage1_body(xv, xu, wv, wu, bv, bu, a_ref, ov, ou):
  a = a_ref[0]
  ev = _prelu(jnp.dot(xv[...], wv[...], preferred_element_type=jnp.float32)
              + bv[...], a)
  eu = _prelu(jnp.dot(xu[...], wu[...], preferred_element_type=jnp.float32)
              + bu[...], a)
  ov[...] = _sc_bf16(ev)
  ou[...] = _sc_bf16(eu)


def _stage2_body(xv, xu, wv, wu, bv, bu, a_ref, wd, ov, ou, oa, oub, ovp, oup):
  a = a_ref[0]
  ev = _prelu(jnp.dot(xv[...], wv[...], preferred_element_type=jnp.float32)
              + bv[...], a)
  eu = _prelu(jnp.dot(xu[...], wu[...], preferred_element_type=jnp.float32)
              + bu[...], a)
  oa[...] = jnp.dot(ev, wd[...],
                    preferred_element_type=jnp.float32).astype(jnp.bfloat16)
  ov[...] = ev
  ou[...] = eu
  oub[...] = eu.astype(jnp.bfloat16)
  ovp[...] = _sc_bf16(ev)
  oup[...] = _sc_bf16(eu)


def _final_body(ev2, eu2, fv, fu, gv, gu, av1, au1,
                wv3, wu3, bv3, bu3, osv, osu):
  c = jnp.float32(1.0 / (DEG + 1.0))
  d = jnp.float32(float(DEG))
  v3 = (jnp.dot(ev2[...], wv3[:D, :], preferred_element_type=jnp.float32)
        + jnp.dot(fv[...], wv3[D:, :], preferred_element_type=jnp.float32)
        + bv3[...])
  u3 = (jnp.dot(eu2[...], wu3[:D, :], preferred_element_type=jnp.float32)
        + jnp.dot(fu[...], wu3[D:, :], preferred_element_type=jnp.float32)
        + bu3[...])
  osv[...] = (jnp.dot(gv[...], wu3[:D, :], preferred_element_type=jnp.float32)
              + d * jnp.dot(av1[...], wu3[D:, :],
                            preferred_element_type=jnp.float32)
              + d * bu3[...] + v3) * c
  osu[...] = (jnp.dot(gu[...], wv3[:D, :], preferred_element_type=jnp.float32)
              + d * jnp.dot(au1[...], wv3[D:, :],
                            preferred_element_type=jnp.float32)
              + d * bv3[...] + u3) * c


def _row_spec():
  return pl.BlockSpec((BM, D), lambda i: (i, 0))


def _packed_spec():
  return pl.BlockSpec((BM, D // 2), lambda i: (i, 0))


def _full_spec():
  return pl.BlockSpec((D, D), lambda i: (0, 0))


def _bias_spec():
  return pl.BlockSpec((1, D), lambda i: (0, 0))


_f32_row = jax.ShapeDtypeStruct((NV, D), jnp.float32)
_bf16_row = jax.ShapeDtypeStruct((NV, D // 2), jnp.int32)

_cast_call = pl.pallas_call(
    _cast_body,
    grid=(NBLK,),
    in_specs=[_row_spec(), _row_spec()],
    out_specs=[_packed_spec(), _packed_spec()],
    out_shape=[_bf16_row, _bf16_row],
)

_stage1_call = pl.pallas_call(
    _stage1_body,
    grid=(NBLK,),
    in_specs=[_row_spec(), _row_spec(), _full_spec(), _full_spec(),
              _bias_spec(), _bias_spec(),
              pl.BlockSpec(memory_space=pltpu.SMEM)],
    out_specs=[_packed_spec(), _packed_spec()],
    out_shape=[_bf16_row, _bf16_row],
)

_stage2_call = pl.pallas_call(
    _stage2_body,
    grid=(NBLK,),
    in_specs=[_row_spec(), _row_spec(), _full_spec(), _full_spec(),
              _bias_spec(), _bias_spec(),
              pl.BlockSpec(memory_space=pltpu.SMEM), _full_spec()],
    out_specs=[_row_spec(), _row_spec(), _row_spec(), _row_spec(),
               _packed_spec(), _packed_spec()],
    out_shape=[_f32_row, _f32_row,
               jax.ShapeDtypeStruct((NV, D), jnp.bfloat16),
               jax.ShapeDtypeStruct((NV, D), jnp.bfloat16),
               _bf16_row, _bf16_row],
)

_final_call = pl.pallas_call(
    _final_body,
    grid=(NBLK,),
    in_specs=[_row_spec(), _row_spec(), _row_spec(), _row_spec(),
              _row_spec(), _row_spec(), _row_spec(), _row_spec(),
              pl.BlockSpec((2 * D, D), lambda i: (0, 0)),
              pl.BlockSpec((2 * D, D), lambda i: (0, 0)),
              _bias_spec(), _bias_spec()],
    out_specs=[_row_spec(), _row_spec()],
    out_shape=[_f32_row, _f32_row],
)


LBM = 512
LBN = 1024
LNI = NV // LBM
LNJ = NV // LBN


def _loss_body(a_ref, u_ref, t_ref, o_ref, acc_ref):
  i = pl.program_id(0)
  j = pl.program_id(1)

  @pl.when((i == 0) & (j == 0))
  def _():
    acc_ref[0] = 0.0
    acc_ref[1] = 0.0
    acc_ref[2] = 0.0
    acc_ref[3] = 0.0

  logit = lax.dot_general(a_ref[...], u_ref[...], (((1,), (1,)), ((), ())),
                          preferred_element_type=jnp.float32)
  lb = logit.astype(jnp.bfloat16)
  sp = (jnp.maximum(lb, jnp.bfloat16(0.0))
        + jnp.log1p(jnp.exp(-jnp.abs(lb)))).astype(jnp.float32)
  tz = t_ref[...] != 0
  acc_ref[0] += jnp.sum(sp)
  acc_ref[1] += jnp.sum(jnp.where(tz, sp - logit, 0.0))
  acc_ref[2] += jnp.sum(jnp.where(tz, sp, 0.0))
  acc_ref[3] += jnp.sum(jnp.where(tz, 1.0, 0.0))

  @pl.when((i == LNI - 1) & (j == LNJ - 1))
  def _():
    n = float(NV) * float(NV)
    s = acc_ref[3]
    norm = n / (n - s)
    pw = (n - s) / s
    val = (norm / n) * (acc_ref[0] + pw * acc_ref[1] - acc_ref[2])
    o_ref[...] = jnp.reshape(val, (1, 1))


_loss_call = pl.pallas_call(
    _loss_body,
    grid=(LNI, LNJ),
    in_specs=[pl.BlockSpec((LBM, D), lambda i, j: (i, 0)),
              pl.BlockSpec((LBN, D), lambda i, j: (j, 0)),
              pl.BlockSpec((LBM, LBN), lambda i, j: (i, j))],
    out_specs=pl.BlockSpec((1, 1), lambda i, j: (0, 0)),
    out_shape=jax.ShapeDtypeStruct((1, 1), jnp.float32),
    scratch_shapes=[pltpu.SMEM((4,), jnp.float32)],
    compiler_params=pltpu.CompilerParams(
        dimension_semantics=("arbitrary", "arbitrary")),
)


def kernel(feat_v, feat_u, neigh_v, neigh_u, target,
           Wv1, bv1, Wu1, bu1, a1, Wv2, bv2, Wu2, bu2, a2,
           Wv3, bv3, Wu3, bu3, Wd):
  iv = neigh_v.reshape(-1)
  iu = neigh_u.reshape(-1)

  fvb, fub = _cast_call(feat_v, feat_u)
  aggv1, aggu1 = _sc_agg_mean(fub, iv, fvb, iu)
  ve1b, ue1b = _stage1_call(aggv1, aggu1, Wv1, Wu1,
                            bv1.reshape(1, D), bu1.reshape(1, D),
                            a1.reshape(1))
  aggv2, aggu2 = _sc_agg_mean(ue1b, iv, ve1b, iu)
  ve2, ue2, Ab, ue2b, ve2p, ue2p = _stage2_call(
      aggv2, aggu2, Wv2, Wu2, bv2.reshape(1, D), bu2.reshape(1, D),
      a2.reshape(1), Wd)
  gv, gu = _sc_agg_sum(ue2p, iv, ve2p, iu)
  loss = _loss_call(Ab, ue2b, target)[0, 0]
  sv, su = _final_call(ve2, ue2, feat_v, feat_u, gv, gu, aggv1, aggu1,
                       Wv3, Wu3, bv3.reshape(1, D), bu3.reshape(1, D))
  return ve2, ue2, sv, su, loss


# speedup vs baseline: 1.1382x; 1.0782x over previous
import functools

import jax
import jax.numpy as jnp
from jax import lax
from jax.experimental import pallas as pl
from jax.experimental.pallas import tpu as pltpu
from jax.experimental.pallas import tpu_sc as plsc

NV = 4096
D = 256
DEG = 16

NC = 2
NS = 16
L = 16
NW = NC * NS
NPW = NV // NW
CH = 16
RPC = CH * DEG
NCHUNK = NPW // CH


def _sc_bf16(x):
  def rne(v):
    u = lax.bitcast_convert_type(v, jnp.int32)
    rb = lax.bitwise_and(lax.shift_right_logical(u, 16), 1)
    return lax.shift_right_logical(u + 32767 + rb, 16)

  return lax.bitwise_or(rne(x[:, :D // 2]),
                        lax.shift_left(rne(x[:, D // 2:]), 16))


@functools.lru_cache(maxsize=None)
def _make_sc_agg(scale: float):
  mesh = plsc.VectorSubcoreMesh(
      core_axis_name="c", subcore_axis_name="s", num_cores=NC, num_subcores=NS)
  out_type = (jax.ShapeDtypeStruct((NV, D), jnp.float32),
              jax.ShapeDtypeStruct((NV, D), jnp.float32))
  scratch = [
      pltpu.VMEM((NPW * DEG,), jnp.int32),
      pltpu.VMEM((NPW * DEG,), jnp.int32),
      pltpu.VMEM((2, RPC, D // 2), jnp.int32),
      pltpu.VMEM((2, CH, D), jnp.float32),
      pltpu.SemaphoreType.DMA,
      pltpu.SemaphoreType.DMA,
      pltpu.SemaphoreType.DMA,
      pltpu.SemaphoreType.DMA,
      pltpu.SemaphoreType.DMA,
  ]
  def body(*refs):
    (ta, ia, tb, ib, outa, outb,
     idx_v0, idx_v1, rows_v, out_v, gsem0, gsem1, osem0, osem1, isem) = refs
    idx_vs = (idx_v0, idx_v1)
    sides = ((ta, ia, outa), (tb, ib, outb))
    gsems = (gsem0, gsem1)
    osems = (osem0, osem1)
    wid = lax.axis_index("s") * NC + lax.axis_index("c")
    node0 = wid * NPW

    for si, (table, idx, out) in enumerate(sides):
      if si == 0:
        pltpu.sync_copy(idx.at[pl.ds(node0 * DEG, NPW * DEG)], idx_v0)
        pltpu.async_copy(sides[1][1].at[pl.ds(node0 * DEG, NPW * DEG)],
                         idx_v1, isem)
      else:
        pltpu.make_async_copy(idx.at[pl.ds(node0 * DEG, NPW * DEG)],
                              idx_v1, isem).wait()

      def start_gather(c, b, table=table, si=si):
        pltpu.async_copy(
            table.at[idx_vs[si].at[pl.ds(c * RPC, RPC)]], rows_v.at[b],
            gsems[b])

      def wait_gather(b, table=table):
        pltpu.make_async_copy(
            table.at[pl.ds(0, RPC)], rows_v.at[b], gsems[b]).wait()

      def wait_out(b, out=out):
        pltpu.make_async_copy(
            out_v.at[b], out.at[pl.ds(node0, CH)], osems[b]).wait()

      def reduce_chunk(c, b):
        mask = jnp.int32(-65536)

        def split(w):
          e = lax.bitcast_convert_type(lax.shift_left(w, 16), jnp.float32)
          o = lax.bitcast_convert_type(lax.bitwise_and(w, mask), jnp.float32)
          return e, o

        def lane_body(g, _):
          sl = pl.ds(g * 16, 16)
          slo = pl.ds(g * 16, 16)
          shi = pl.ds(D // 2 + g * 16, 16)
          for cc in range(CH):
            e, o = split(rows_v[b, cc * DEG, sl])
            for r in range(1, DEG):
              e2, o2 = split(rows_v[b, cc * DEG + r, sl])
              e = e + e2
              o = o + o2
            out_v[b, cc, slo] = e * scale
            out_v[b, cc, shi] = o * scale
          return 0
        lax.fori_loop(0, D // 32, lane_body, 0)

      start_gather(0, 0)

      def pair_body(kk, _, out=out):
        for b in range(2):
          c = kk * 2 + b
          wait_gather(b)

          @pl.when(c + 1 < NCHUNK)
          def _():
            start_gather(c + 1, 1 - b)

          @pl.when(c >= 2)
          def _():
            wait_out(b)

          reduce_chunk(c, b)
          pltpu.async_copy(out_v.at[b],
                           out.at[pl.ds(node0 + c * CH, CH)], osems[b])
        return 0

      lax.fori_loop(0, NCHUNK // 2, pair_body, 0)
      for b in range(2):
        wait_out(b)

  return pl.kernel(body, out_type=out_type, mesh=mesh, scratch_types=scratch)


def _sc_agg_mean(*args):
  return _make_sc_agg(1.0 / DEG)(*args)


def _sc_agg_sum(*args):
  return _make_sc_agg(1.0)(*args)


BM = 512
NBLK = NV // BM


def _prelu(y, a):
  return jnp.where(y >= 0, y, a * y)


def _cast_body(xv, xu, ov, ou):
  ov[...] = _sc_bf16(xv[...])
  ou[...] = _sc_bf16(xu[...])


def _stage1_body(xv, xu, wv, wu, bv, bu, a_ref, ov, ou):
  a = a_ref[0]
  ev = _prelu(jnp.dot(xv[...], wv[...], preferred_element_type=jnp.float32)
              + bv[...], a)
  eu = _prelu(jnp.dot(xu[...], wu[...], preferred_element_type=jnp.float32)
              + bu[...], a)
  ov[...] = _sc_bf16(ev)
  ou[...] = _sc_bf16(eu)


def _stage2_body(xv, xu, wv, wu, bv, bu, a_ref, wd, ov, ou, oa, oub, ovp, oup):
  a = a_ref[0]
  ev = _prelu(jnp.dot(xv[...], wv[...], preferred_element_type=jnp.float32)
              + bv[...], a)
  eu = _prelu(jnp.dot(xu[...], wu[...], preferred_element_type=jnp.float32)
              + bu[...], a)
  oa[...] = jnp.dot(ev, wd[...],
                    preferred_element_type=jnp.float32).astype(jnp.bfloat16)
  ov[...] = ev
  ou[...] = eu
  oub[...] = eu.astype(jnp.bfloat16)
  ovp[...] = _sc_bf16(ev)
  oup[...] = _sc_bf16(eu)


def _final_body(ev2, eu2, fv, fu, gv, gu, av1, au1,
                wv3, wu3, bv3, bu3, osv, osu):
  c = jnp.float32(1.0 / (DEG + 1.0))
  d = jnp.float32(float(DEG))
  v3 = (jnp.dot(ev2[...], wv3[:D, :], preferred_element_type=jnp.float32)
        + jnp.dot(fv[...], wv3[D:, :], preferred_element_type=jnp.float32)
        + bv3[...])
  u3 = (jnp.dot(eu2[...], wu3[:D, :], preferred_element_type=jnp.float32)
        + jnp.dot(fu[...], wu3[D:, :], preferred_element_type=jnp.float32)
        + bu3[...])
  osv[...] = (jnp.dot(gv[...], wu3[:D, :], preferred_element_type=jnp.float32)
              + d * jnp.dot(av1[...], wu3[D:, :],
                            preferred_element_type=jnp.float32)
              + d * bu3[...] + v3) * c
  osu[...] = (jnp.dot(gu[...], wv3[:D, :], preferred_element_type=jnp.float32)
              + d * jnp.dot(au1[...], wv3[D:, :],
                            preferred_element_type=jnp.float32)
              + d * bv3[...] + u3) * c


def _row_spec():
  return pl.BlockSpec((BM, D), lambda i: (i, 0))


def _packed_spec():
  return pl.BlockSpec((BM, D // 2), lambda i: (i, 0))


def _full_spec():
  return pl.BlockSpec((D, D), lambda i: (0, 0))


def _bias_spec():
  return pl.BlockSpec((1, D), lambda i: (0, 0))


_f32_row = jax.ShapeDtypeStruct((NV, D), jnp.float32)
_bf16_row = jax.ShapeDtypeStruct((NV, D // 2), jnp.int32)

_cast_call = pl.pallas_call(
    _cast_body,
    grid=(NBLK,),
    in_specs=[_row_spec(), _row_spec()],
    out_specs=[_packed_spec(), _packed_spec()],
    out_shape=[_bf16_row, _bf16_row],
)

_stage1_call = pl.pallas_call(
    _stage1_body,
    grid=(NBLK,),
    in_specs=[_row_spec(), _row_spec(), _full_spec(), _full_spec(),
              _bias_spec(), _bias_spec(),
              pl.BlockSpec(memory_space=pltpu.SMEM)],
    out_specs=[_packed_spec(), _packed_spec()],
    out_shape=[_bf16_row, _bf16_row],
)

_stage2_call = pl.pallas_call(
    _stage2_body,
    grid=(NBLK,),
    in_specs=[_row_spec(), _row_spec(), _full_spec(), _full_spec(),
              _bias_spec(), _bias_spec(),
              pl.BlockSpec(memory_space=pltpu.SMEM), _full_spec()],
    out_specs=[_row_spec(), _row_spec(), _row_spec(), _row_spec(),
               _packed_spec(), _packed_spec()],
    out_shape=[_f32_row, _f32_row,
               jax.ShapeDtypeStruct((NV, D), jnp.bfloat16),
               jax.ShapeDtypeStruct((NV, D), jnp.bfloat16),
               _bf16_row, _bf16_row],
)

_final_call = pl.pallas_call(
    _final_body,
    grid=(NBLK,),
    in_specs=[_row_spec(), _row_spec(), _row_spec(), _row_spec(),
              _row_spec(), _row_spec(), _row_spec(), _row_spec(),
              pl.BlockSpec((2 * D, D), lambda i: (0, 0)),
              pl.BlockSpec((2 * D, D), lambda i: (0, 0)),
              _bias_spec(), _bias_spec()],
    out_specs=[_row_spec(), _row_spec()],
    out_shape=[_f32_row, _f32_row],
)


LBM = 512
LBN = 1024
LNI = NV // LBM
LNJ = NV // LBN


def _loss_body(a_ref, u_ref, t_ref, o_ref, acc_ref):
  i = pl.program_id(0)
  j = pl.program_id(1)

  @pl.when((i == 0) & (j == 0))
  def _():
    acc_ref[0] = 0.0
    acc_ref[1] = 0.0
    acc_ref[2] = 0.0
    acc_ref[3] = 0.0

  logit = lax.dot_general(a_ref[...], u_ref[...], (((1,), (1,)), ((), ())),
                          preferred_element_type=jnp.float32)
  lb = logit.astype(jnp.bfloat16)
  sp = (jnp.maximum(lb, jnp.bfloat16(0.0))
        + jnp.log1p(jnp.exp(-jnp.abs(lb)))).astype(jnp.float32)
  tz = t_ref[...] != 0
  acc_ref[0] += jnp.sum(sp)
  acc_ref[1] += jnp.sum(jnp.where(tz, sp - logit, 0.0))
  acc_ref[2] += jnp.sum(jnp.where(tz, sp, 0.0))
  acc_ref[3] += jnp.sum(jnp.where(tz, 1.0, 0.0))

  @pl.when((i == LNI - 1) & (j == LNJ - 1))
  def _():
    n = float(NV) * float(NV)
    s = acc_ref[3]
    norm = n / (n - s)
    pw = (n - s) / s
    val = (norm / n) * (acc_ref[0] + pw * acc_ref[1] - acc_ref[2])
    o_ref[...] = jnp.reshape(val, (1, 1))


_loss_call = pl.pallas_call(
    _loss_body,
    grid=(LNI, LNJ),
    in_specs=[pl.BlockSpec((LBM, D), lambda i, j: (i, 0)),
              pl.BlockSpec((LBN, D), lambda i, j: (j, 0)),
              pl.BlockSpec((LBM, LBN), lambda i, j: (i, j))],
    out_specs=pl.BlockSpec((1, 1), lambda i, j: (0, 0)),
    out_shape=jax.ShapeDtypeStruct((1, 1), jnp.float32),
    scratch_shapes=[pltpu.SMEM((4,), jnp.float32)],
    compiler_params=pltpu.CompilerParams(
        dimension_semantics=("arbitrary", "arbitrary")),
)


def kernel(feat_v, feat_u, neigh_v, neigh_u, target,
           Wv1, bv1, Wu1, bu1, a1, Wv2, bv2, Wu2, bu2, a2,
           Wv3, bv3, Wu3, bu3, Wd):
  iv = neigh_v.reshape(-1)
  iu = neigh_u.reshape(-1)

  fvb, fub = _cast_call(feat_v, feat_u)
  aggv1, aggu1 = _sc_agg_mean(fub, iv, fvb, iu)
  ve1b, ue1b = _stage1_call(aggv1, aggu1, Wv1, Wu1,
                            bv1.reshape(1, D), bu1.reshape(1, D),
                            a1.reshape(1))
  aggv2, aggu2 = _sc_agg_mean(ue1b, iv, ve1b, iu)
  ve2, ue2, Ab, ue2b, ve2p, ue2p = _stage2_call(
      aggv2, aggu2, Wv2, Wu2, bv2.reshape(1, D), bu2.reshape(1, D),
      a2.reshape(1), Wd)
  gv, gu = _sc_agg_sum(ue2p, iv, ve2p, iu)
  loss = _loss_call(Ab, ue2b, target)[0, 0]
  sv, su = _final_call(ve2, ue2, feat_v, feat_u, gv, gu, aggv1, aggu1,
                       Wv3, Wu3, bv3.reshape(1, D), bu3.reshape(1, D))
  return ve2, ue2, sv, su, loss


# speedup vs baseline: 1.1683x; 1.0264x over previous
import functools

import jax
import jax.numpy as jnp
from jax import lax
from jax.experimental import pallas as pl
from jax.experimental.pallas import tpu as pltpu
from jax.experimental.pallas import tpu_sc as plsc

NV = 4096
D = 256
DEG = 16

NC = 2
NS = 16
L = 16
NW = NC * NS
NPW = NV // NW
CH = 16
RPC = CH * DEG
NCHUNK = NPW // CH


def _sc_bf16(x):
  def rne(v):
    u = lax.bitcast_convert_type(v, jnp.int32)
    rb = lax.bitwise_and(lax.shift_right_logical(u, 16), 1)
    return lax.shift_right_logical(u + 32767 + rb, 16)

  return lax.bitwise_or(rne(x[:, :D // 2]),
                        lax.shift_left(rne(x[:, D // 2:]), 16))


@functools.lru_cache(maxsize=None)
def _make_sc_agg(scale: float):
  mesh = plsc.VectorSubcoreMesh(
      core_axis_name="c", subcore_axis_name="s", num_cores=NC, num_subcores=NS)
  out_type = (jax.ShapeDtypeStruct((NV, D), jnp.float32),
              jax.ShapeDtypeStruct((NV, D), jnp.float32))
  scratch = [
      pltpu.VMEM((NPW * DEG,), jnp.int32),
      pltpu.VMEM((NPW * DEG,), jnp.int32),
      pltpu.VMEM((2, RPC, D // 2), jnp.int32),
      pltpu.VMEM((2, CH, D), jnp.float32),
      pltpu.SemaphoreType.DMA,
      pltpu.SemaphoreType.DMA,
      pltpu.SemaphoreType.DMA,
      pltpu.SemaphoreType.DMA,
      pltpu.SemaphoreType.DMA,
  ]
  def body(*refs):
    (ta, ia, tb, ib, outa, outb,
     idx_v0, idx_v1, rows_v, out_v, gsem0, gsem1, osem0, osem1, isem) = refs
    idx_vs = (idx_v0, idx_v1)
    sides = ((ta, ia, outa), (tb, ib, outb))
    gsems = (gsem0, gsem1)
    osems = (osem0, osem1)
    wid = lax.axis_index("s") * NC + lax.axis_index("c")
    node0 = wid * NPW

    for si, (table, idx, out) in enumerate(sides):
      if si == 0:
        pltpu.sync_copy(idx.at[pl.ds(node0 * DEG, NPW * DEG)], idx_v0)
        pltpu.async_copy(sides[1][1].at[pl.ds(node0 * DEG, NPW * DEG)],
                         idx_v1, isem)
      else:
        pltpu.make_async_copy(idx.at[pl.ds(node0 * DEG, NPW * DEG)],
                              idx_v1, isem).wait()

      def start_gather(c, b, table=table, si=si):
        pltpu.async_copy(
            table.at[idx_vs[si].at[pl.ds(c * RPC, RPC)]], rows_v.at[b],
            gsems[b])

      def wait_gather(b, table=table):
        pltpu.make_async_copy(
            table.at[pl.ds(0, RPC)], rows_v.at[b], gsems[b]).wait()

      def wait_out(b, out=out):
        pltpu.make_async_copy(
            out_v.at[b], out.at[pl.ds(node0, CH)], osems[b]).wait()

      def reduce_chunk(c, b):
        mask = jnp.int32(-65536)

        def split(w):
          e = lax.bitcast_convert_type(lax.shift_left(w, 16), jnp.float32)
          o = lax.bitcast_convert_type(lax.bitwise_and(w, mask), jnp.float32)
          return e, o

        def lane_body(g, _):
          sl = pl.ds(g * 16, 16)
          slo = pl.ds(g * 16, 16)
          shi = pl.ds(D // 2 + g * 16, 16)
          for cc in range(CH):
            e, o = split(rows_v[b, cc * DEG, sl])
            for r in range(1, DEG):
              e2, o2 = split(rows_v[b, cc * DEG + r, sl])
              e = e + e2
              o = o + o2
            out_v[b, cc, slo] = e * scale
            out_v[b, cc, shi] = o * scale
          return 0
        lax.fori_loop(0, D // 32, lane_body, 0)

      start_gather(0, 0)

      def pair_body(kk, _, out=out):
        for b in range(2):
          c = kk * 2 + b
          wait_gather(b)

          @pl.when(c + 1 < NCHUNK)
          def _():
            start_gather(c + 1, 1 - b)

          @pl.when(c >= 2)
          def _():
            wait_out(b)

          reduce_chunk(c, b)
          pltpu.async_copy(out_v.at[b],
                           out.at[pl.ds(node0 + c * CH, CH)], osems[b])
        return 0

      lax.fori_loop(0, NCHUNK // 2, pair_body, 0)
      for b in range(2):
        wait_out(b)

  return pl.kernel(body, out_type=out_type, mesh=mesh, scratch_types=scratch)


def _sc_agg_mean(*args):
  return _make_sc_agg(1.0 / DEG)(*args)


def _sc_agg_sum(*args):
  return _make_sc_agg(1.0)(*args)


BM = 512
NBLK = NV // BM


def _prelu(y, a):
  return jnp.where(y >= 0, y, a * y)


def _cast_body(xv, xu, ov, ou):
  ov[...] = _sc_bf16(xv[...])
  ou[...] = _sc_bf16(xu[...])


def _stage1_body(xv, xu, wv, wu, bv, bu, a_ref, ov, ou):
  a = a_ref[0]
  ev = _prelu(jnp.dot(xv[...], wv[...], preferred_element_type=jnp.float32)
              + bv[...], a)
  eu = _prelu(jnp.dot(xu[...], wu[...], preferred_element_type=jnp.float32)
              + bu[...], a)
  ov[...] = _sc_bf16(ev)
  ou[...] = _sc_bf16(eu)


def _stage2_body(xv, xu, wv, wu, bv, bu, a_ref, wd, ov, ou, oa, oub, ovp, oup):
  a = a_ref[0]
  ev = _prelu(jnp.dot(xv[...], wv[...], preferred_element_type=jnp.float32)
              + bv[...], a)
  eu = _prelu(jnp.dot(xu[...], wu[...], preferred_element_type=jnp.float32)
              + bu[...], a)
  oa[...] = jnp.dot(ev, wd[...],
                    preferred_element_type=jnp.float32).astype(jnp.bfloat16)
  ov[...] = ev
  ou[...] = eu
  oub[...] = eu.astype(jnp.bfloat16)
  ovp[...] = _sc_bf16(ev)
  oup[...] = _sc_bf16(eu)


def _final_body(ev2, eu2, fv, fu, gv, gu, av1, au1,
                wv3, wu3, bv3, bu3, osv, osu):
  c = jnp.float32(1.0 / (DEG + 1.0))
  d = jnp.float32(float(DEG))
  v3 = (jnp.dot(ev2[...], wv3[:D, :], preferred_element_type=jnp.float32)
        + jnp.dot(fv[...], wv3[D:, :], preferred_element_type=jnp.float32)
        + bv3[...])
  u3 = (jnp.dot(eu2[...], wu3[:D, :], preferred_element_type=jnp.float32)
        + jnp.dot(fu[...], wu3[D:, :], preferred_element_type=jnp.float32)
        + bu3[...])
  osv[...] = (jnp.dot(gv[...], wu3[:D, :], preferred_element_type=jnp.float32)
              + d * jnp.dot(av1[...], wu3[D:, :],
                            preferred_element_type=jnp.float32)
              + d * bu3[...] + v3) * c
  osu[...] = (jnp.dot(gu[...], wv3[:D, :], preferred_element_type=jnp.float32)
              + d * jnp.dot(au1[...], wv3[D:, :],
                            preferred_element_type=jnp.float32)
              + d * bv3[...] + u3) * c


def _row_spec():
  return pl.BlockSpec((BM, D), lambda i: (i, 0))


def _packed_spec():
  return pl.BlockSpec((BM, D // 2), lambda i: (i, 0))


def _full_spec():
  return pl.BlockSpec((D, D), lambda i: (0, 0))


def _bias_spec():
  return pl.BlockSpec((D,), lambda i: (0,))


_f32_row = jax.ShapeDtypeStruct((NV, D), jnp.float32)
_bf16_row = jax.ShapeDtypeStruct((NV, D // 2), jnp.int32)

_cast_call = pl.pallas_call(
    _cast_body,
    grid=(NBLK,),
    in_specs=[_row_spec(), _row_spec()],
    out_specs=[_packed_spec(), _packed_spec()],
    out_shape=[_bf16_row, _bf16_row],
)

_stage1_call = pl.pallas_call(
    _stage1_body,
    grid=(NBLK,),
    in_specs=[_row_spec(), _row_spec(), _full_spec(), _full_spec(),
              _bias_spec(), _bias_spec(),
              pl.BlockSpec(memory_space=pltpu.SMEM)],
    out_specs=[_packed_spec(), _packed_spec()],
    out_shape=[_bf16_row, _bf16_row],
)

_stage2_call = pl.pallas_call(
    _stage2_body,
    grid=(NBLK,),
    in_specs=[_row_spec(), _row_spec(), _full_spec(), _full_spec(),
              _bias_spec(), _bias_spec(),
              pl.BlockSpec(memory_space=pltpu.SMEM), _full_spec()],
    out_specs=[_row_spec(), _row_spec(), _row_spec(), _row_spec(),
               _packed_spec(), _packed_spec()],
    out_shape=[_f32_row, _f32_row,
               jax.ShapeDtypeStruct((NV, D), jnp.bfloat16),
               jax.ShapeDtypeStruct((NV, D), jnp.bfloat16),
               _bf16_row, _bf16_row],
)

_final_call = pl.pallas_call(
    _final_body,
    grid=(NBLK,),
    in_specs=[_row_spec(), _row_spec(), _row_spec(), _row_spec(),
              _row_spec(), _row_spec(), _row_spec(), _row_spec(),
              pl.BlockSpec((2 * D, D), lambda i: (0, 0)),
              pl.BlockSpec((2 * D, D), lambda i: (0, 0)),
              _bias_spec(), _bias_spec()],
    out_specs=[_row_spec(), _row_spec()],
    out_shape=[_f32_row, _f32_row],
)


LBM = 512
LBN = 2048
LNI = NV // LBM
LNJ = NV // LBN


def _loss_body(a_ref, u_ref, t_ref, o_ref, acc_ref):
  i = pl.program_id(0)
  j = pl.program_id(1)

  @pl.when((i == 0) & (j == 0))
  def _():
    acc_ref[0] = 0.0
    acc_ref[1] = 0.0
    acc_ref[2] = 0.0
    acc_ref[3] = 0.0

  logit = lax.dot_general(a_ref[...], u_ref[...], (((1,), (1,)), ((), ())),
                          preferred_element_type=jnp.float32)
  lb = logit.astype(jnp.bfloat16)
  sp = (jnp.maximum(lb, jnp.bfloat16(0.0))
        + jnp.log1p(jnp.exp(-jnp.abs(lb)))).astype(jnp.float32)
  tz = t_ref[...] != 0
  acc_ref[0] += jnp.sum(sp)
  acc_ref[1] += jnp.sum(jnp.where(tz, sp - logit, 0.0))
  acc_ref[2] += jnp.sum(jnp.where(tz, sp, 0.0))
  acc_ref[3] += jnp.sum(jnp.where(tz, 1.0, 0.0))

  @pl.when((i == LNI - 1) & (j == LNJ - 1))
  def _():
    n = float(NV) * float(NV)
    s = acc_ref[3]
    norm = n / (n - s)
    pw = (n - s) / s
    val = (norm / n) * (acc_ref[0] + pw * acc_ref[1] - acc_ref[2])
    o_ref[...] = jnp.reshape(val, (1, 1))


_loss_call = pl.pallas_call(
    _loss_body,
    grid=(LNI, LNJ),
    in_specs=[pl.BlockSpec((LBM, D), lambda i, j: (i, 0)),
              pl.BlockSpec((LBN, D), lambda i, j: (j, 0)),
              pl.BlockSpec((LBM, LBN), lambda i, j: (i, j))],
    out_specs=pl.BlockSpec((1, 1), lambda i, j: (0, 0)),
    out_shape=jax.ShapeDtypeStruct((1, 1), jnp.float32),
    scratch_shapes=[pltpu.SMEM((4,), jnp.float32)],
    compiler_params=pltpu.CompilerParams(
        dimension_semantics=("arbitrary", "arbitrary")),
)


def kernel(feat_v, feat_u, neigh_v, neigh_u, target,
           Wv1, bv1, Wu1, bu1, a1, Wv2, bv2, Wu2, bu2, a2,
           Wv3, bv3, Wu3, bu3, Wd):
  iv = neigh_v.reshape(-1)
  iu = neigh_u.reshape(-1)

  fvb, fub = _cast_call(feat_v, feat_u)
  aggv1, aggu1 = _sc_agg_mean(fub, iv, fvb, iu)
  ve1b, ue1b = _stage1_call(aggv1, aggu1, Wv1, Wu1,
                            bv1, bu1,
                            a1.reshape(1))
  aggv2, aggu2 = _sc_agg_mean(ue1b, iv, ve1b, iu)
  ve2, ue2, Ab, ue2b, ve2p, ue2p = _stage2_call(
      aggv2, aggu2, Wv2, Wu2, bv2, bu2,
      a2.reshape(1), Wd)
  gv, gu = _sc_agg_sum(ue2p, iv, ve2p, iu)
  loss = _loss_call(Ab, ue2b, target)[0, 0]
  sv, su = _final_call(ve2, ue2, feat_v, feat_u, gv, gu, aggv1, aggu1,
                       Wv3, Wu3, bv3, bu3)
  return ve2, ue2, sv, su, loss


# speedup vs baseline: 1.1776x; 1.0080x over previous
import functools

import jax
import jax.numpy as jnp
from jax import lax
from jax.experimental import pallas as pl
from jax.experimental.pallas import tpu as pltpu
from jax.experimental.pallas import tpu_sc as plsc

NV = 4096
D = 256
DEG = 16

NC = 2
NS = 16
L = 16
NW = NC * NS
NPW = NV // NW
CH = 16
RPC = CH * DEG
NCHUNK = NPW // CH


def _sc_bf16(x):
  def rne(v):
    u = lax.bitcast_convert_type(v, jnp.int32)
    rb = lax.bitwise_and(lax.shift_right_logical(u, 16), 1)
    return lax.shift_right_logical(u + 32767 + rb, 16)

  return lax.bitwise_or(rne(x[:, :D // 2]),
                        lax.shift_left(rne(x[:, D // 2:]), 16))


@functools.lru_cache(maxsize=None)
def _make_sc_agg(scale: float, exact: bool):
  mesh = plsc.VectorSubcoreMesh(
      core_axis_name="c", subcore_axis_name="s", num_cores=NC, num_subcores=NS)
  out_type = (jax.ShapeDtypeStruct((NV, D), jnp.float32),
              jax.ShapeDtypeStruct((NV, D), jnp.float32))
  scratch = [
      pltpu.VMEM((NPW * DEG,), jnp.int32),
      pltpu.VMEM((NPW * DEG,), jnp.int32),
      pltpu.VMEM((2, RPC, D // 2), jnp.int32),
      pltpu.VMEM((2, CH, D), jnp.float32),
      pltpu.SemaphoreType.DMA,
      pltpu.SemaphoreType.DMA,
      pltpu.SemaphoreType.DMA,
      pltpu.SemaphoreType.DMA,
      pltpu.SemaphoreType.DMA,
  ]
  def body(*refs):
    (ta, ia, tb, ib, outa, outb,
     idx_v0, idx_v1, rows_v, out_v, gsem0, gsem1, osem0, osem1, isem) = refs
    idx_vs = (idx_v0, idx_v1)
    sides = ((ta, ia, outa), (tb, ib, outb))
    gsems = (gsem0, gsem1)
    osems = (osem0, osem1)
    wid = lax.axis_index("s") * NC + lax.axis_index("c")
    node0 = wid * NPW

    for si, (table, idx, out) in enumerate(sides):
      if si == 0:
        pltpu.sync_copy(idx.at[pl.ds(node0 * DEG, NPW * DEG)], idx_v0)
        pltpu.async_copy(sides[1][1].at[pl.ds(node0 * DEG, NPW * DEG)],
                         idx_v1, isem)
      else:
        pltpu.make_async_copy(idx.at[pl.ds(node0 * DEG, NPW * DEG)],
                              idx_v1, isem).wait()

      def start_gather(c, b, table=table, si=si):
        pltpu.async_copy(
            table.at[idx_vs[si].at[pl.ds(c * RPC, RPC)]], rows_v.at[b],
            gsems[b])

      def wait_gather(b, table=table):
        pltpu.make_async_copy(
            table.at[pl.ds(0, RPC)], rows_v.at[b], gsems[b]).wait()

      def wait_out(b, out=out):
        pltpu.make_async_copy(
            out_v.at[b], out.at[pl.ds(node0, CH)], osems[b]).wait()

      def reduce_chunk(c, b):
        mask = jnp.int32(-65536)

        def split(w):
          e = lax.bitcast_convert_type(lax.shift_left(w, 16), jnp.float32)
          if exact:
            o = lax.bitcast_convert_type(lax.bitwise_and(w, mask), jnp.float32)
          else:
            o = lax.bitcast_convert_type(w, jnp.float32)
          return e, o

        def lane_body(g, _):
          sl = pl.ds(g * 16, 16)
          slo = pl.ds(g * 16, 16)
          shi = pl.ds(D // 2 + g * 16, 16)
          for cc in range(CH):
            e, o = split(rows_v[b, cc * DEG, sl])
            for r in range(1, DEG):
              e2, o2 = split(rows_v[b, cc * DEG + r, sl])
              e = e + e2
              o = o + o2
            out_v[b, cc, slo] = e * scale
            out_v[b, cc, shi] = o * scale
          return 0
        lax.fori_loop(0, D // 32, lane_body, 0)

      start_gather(0, 0)

      def pair_body(kk, _, out=out):
        for b in range(2):
          c = kk * 2 + b
          wait_gather(b)

          @pl.when(c + 1 < NCHUNK)
          def _():
            start_gather(c + 1, 1 - b)

          @pl.when(c >= 2)
          def _():
            wait_out(b)

          reduce_chunk(c, b)
          pltpu.async_copy(out_v.at[b],
                           out.at[pl.ds(node0 + c * CH, CH)], osems[b])
        return 0

      lax.fori_loop(0, NCHUNK // 2, pair_body, 0)
      for b in range(2):
        wait_out(b)

  return pl.kernel(body, out_type=out_type, mesh=mesh, scratch_types=scratch)


def _sc_agg_mean(*args):
  return _make_sc_agg(1.0 / DEG, False)(*args)


def _sc_agg_sum(*args):
  return _make_sc_agg(1.0, True)(*args)


BM = 512
NBLK = NV // BM


def _prelu(y, a):
  return jnp.where(y >= 0, y, a * y)


def _cast_body(xv, xu, ov, ou):
  ov[...] = _sc_bf16(xv[...])
  ou[...] = _sc_bf16(xu[...])


def _stage1_body(xv, xu, wv, wu, bv, bu, a_ref, ov, ou):
  a = a_ref[0]
  ev = _prelu(jnp.dot(xv[...], wv[...], preferred_element_type=jnp.float32)
              + bv[...], a)
  eu = _prelu(jnp.dot(xu[...], wu[...], preferred_element_type=jnp.float32)
              + bu[...], a)
  ov[...] = _sc_bf16(ev)
  ou[...] = _sc_bf16(eu)


def _stage2_body(xv, xu, wv, wu, bv, bu, a_ref, wd, ov, ou, oa, oub, ovp, oup):
  a = a_ref[0]
  ev = _prelu(jnp.dot(xv[...], wv[...], preferred_element_type=jnp.float32)
              + bv[...], a)
  eu = _prelu(jnp.dot(xu[...], wu[...], preferred_element_type=jnp.float32)
              + bu[...], a)
  oa[...] = jnp.dot(ev, wd[...],
                    preferred_element_type=jnp.float32).astype(jnp.bfloat16)
  ov[...] = ev
  ou[...] = eu
  oub[...] = eu.astype(jnp.bfloat16)
  ovp[...] = _sc_bf16(ev)
  oup[...] = _sc_bf16(eu)


def _final_body(ev2, eu2, fv, fu, gv, gu, av1, au1,
                wv3, wu3, bv3, bu3, osv, osu):
  c = jnp.float32(1.0 / (DEG + 1.0))
  d = jnp.float32(float(DEG))
  v3 = (jnp.dot(ev2[...], wv3[:D, :], preferred_element_type=jnp.float32)
        + jnp.dot(fv[...], wv3[D:, :], preferred_element_type=jnp.float32)
        + bv3[...])
  u3 = (jnp.dot(eu2[...], wu3[:D, :], preferred_element_type=jnp.float32)
        + jnp.dot(fu[...], wu3[D:, :], preferred_element_type=jnp.float32)
        + bu3[...])
  osv[...] = (jnp.dot(gv[...], wu3[:D, :], preferred_element_type=jnp.float32)
              + d * jnp.dot(av1[...], wu3[D:, :],
                            preferred_element_type=jnp.float32)
              + d * bu3[...] + v3) * c
  osu[...] = (jnp.dot(gu[...], wv3[:D, :], preferred_element_type=jnp.float32)
              + d * jnp.dot(au1[...], wv3[D:, :],
                            preferred_element_type=jnp.float32)
              + d * bv3[...] + u3) * c


def _row_spec():
  return pl.BlockSpec((BM, D), lambda i: (i, 0))


def _packed_spec():
  return pl.BlockSpec((BM, D // 2), lambda i: (i, 0))


def _full_spec():
  return pl.BlockSpec((D, D), lambda i: (0, 0))


def _bias_spec():
  return pl.BlockSpec((D,), lambda i: (0,))


_f32_row = jax.ShapeDtypeStruct((NV, D), jnp.float32)
_bf16_row = jax.ShapeDtypeStruct((NV, D // 2), jnp.int32)

_cast_call = pl.pallas_call(
    _cast_body,
    grid=(NBLK,),
    in_specs=[_row_spec(), _row_spec()],
    out_specs=[_packed_spec(), _packed_spec()],
    out_shape=[_bf16_row, _bf16_row],
)

_stage1_call = pl.pallas_call(
    _stage1_body,
    grid=(NBLK,),
    in_specs=[_row_spec(), _row_spec(), _full_spec(), _full_spec(),
              _bias_spec(), _bias_spec(),
              pl.BlockSpec(memory_space=pltpu.SMEM)],
    out_specs=[_packed_spec(), _packed_spec()],
    out_shape=[_bf16_row, _bf16_row],
)

_stage2_call = pl.pallas_call(
    _stage2_body,
    grid=(NBLK,),
    in_specs=[_row_spec(), _row_spec(), _full_spec(), _full_spec(),
              _bias_spec(), _bias_spec(),
              pl.BlockSpec(memory_space=pltpu.SMEM), _full_spec()],
    out_specs=[_row_spec(), _row_spec(), _row_spec(), _row_spec(),
               _packed_spec(), _packed_spec()],
    out_shape=[_f32_row, _f32_row,
               jax.ShapeDtypeStruct((NV, D), jnp.bfloat16),
               jax.ShapeDtypeStruct((NV, D), jnp.bfloat16),
               _bf16_row, _bf16_row],
)

_final_call = pl.pallas_call(
    _final_body,
    grid=(NBLK,),
    in_specs=[_row_spec(), _row_spec(), _row_spec(), _row_spec(),
              _row_spec(), _row_spec(), _row_spec(), _row_spec(),
              pl.BlockSpec((2 * D, D), lambda i: (0, 0)),
              pl.BlockSpec((2 * D, D), lambda i: (0, 0)),
              _bias_spec(), _bias_spec()],
    out_specs=[_row_spec(), _row_spec()],
    out_shape=[_f32_row, _f32_row],
)


LBM = 512
LBN = 2048
LNI = NV // LBM
LNJ = NV // LBN


def _loss_body(a_ref, u_ref, t_ref, o_ref, acc_ref):
  i = pl.program_id(0)
  j = pl.program_id(1)

  @pl.when((i == 0) & (j == 0))
  def _():
    acc_ref[0] = 0.0
    acc_ref[1] = 0.0
    acc_ref[2] = 0.0
    acc_ref[3] = 0.0

  logit = lax.dot_general(a_ref[...], u_ref[...], (((1,), (1,)), ((), ())),
                          preferred_element_type=jnp.float32)
  lb = logit.astype(jnp.bfloat16)
  sp = (jnp.maximum(lb, jnp.bfloat16(0.0))
        + jnp.log1p(jnp.exp(-jnp.abs(lb)))).astype(jnp.float32)
  tz = t_ref[...] != 0
  acc_ref[0] += jnp.sum(sp)
  acc_ref[1] += jnp.sum(jnp.where(tz, logit, 0.0))
  acc_ref[2] += jnp.sum(jnp.where(tz, sp, 0.0))
  acc_ref[3] += jnp.sum(jnp.where(tz, 1.0, 0.0))

  @pl.when((i == LNI - 1) & (j == LNJ - 1))
  def _():
    n = float(NV) * float(NV)
    s = acc_ref[3]
    norm = n / (n - s)
    pw = (n - s) / s
    val = (norm / n) * (acc_ref[0] + pw * (acc_ref[2] - acc_ref[1])
                        - acc_ref[2])
    o_ref[...] = jnp.reshape(val, (1, 1))


_loss_call = pl.pallas_call(
    _loss_body,
    grid=(LNI, LNJ),
    in_specs=[pl.BlockSpec((LBM, D), lambda i, j: (i, 0)),
              pl.BlockSpec((LBN, D), lambda i, j: (j, 0)),
              pl.BlockSpec((LBM, LBN), lambda i, j: (i, j))],
    out_specs=pl.BlockSpec((1, 1), lambda i, j: (0, 0)),
    out_shape=jax.ShapeDtypeStruct((1, 1), jnp.float32),
    scratch_shapes=[pltpu.SMEM((4,), jnp.float32)],
    compiler_params=pltpu.CompilerParams(
        dimension_semantics=("arbitrary", "arbitrary")),
)


def kernel(feat_v, feat_u, neigh_v, neigh_u, target,
           Wv1, bv1, Wu1, bu1, a1, Wv2, bv2, Wu2, bu2, a2,
           Wv3, bv3, Wu3, bu3, Wd):
  iv = neigh_v.reshape(-1)
  iu = neigh_u.reshape(-1)

  fvb, fub = _cast_call(feat_v, feat_u)
  aggv1, aggu1 = _sc_agg_mean(fub, iv, fvb, iu)
  ve1b, ue1b = _stage1_call(aggv1, aggu1, Wv1, Wu1,
                            bv1, bu1,
                            a1.reshape(1))
  aggv2, aggu2 = _sc_agg_mean(ue1b, iv, ve1b, iu)
  ve2, ue2, Ab, ue2b, ve2p, ue2p = _stage2_call(
      aggv2, aggu2, Wv2, Wu2, bv2, bu2,
      a2.reshape(1), Wd)
  gv, gu = _sc_agg_sum(ue2p, iv, ve2p, iu)
  loss = _loss_call(Ab, ue2b, target)[0, 0]
  sv, su = _final_call(ve2, ue2, feat_v, feat_u, gv, gu, aggv1, aggu1,
                       Wv3, Wu3, bv3, bu3)
  return ve2, ue2, sv, su, loss
